# baseline (speedup 1.0000x reference)
import jax
import jax.numpy as jnp
from jax.experimental import pallas as pl
import jax.experimental.pallas.tpu as pltpu

_B = 16
_C = 768
_P = 512
_HW = 1024
_SPLIT = 2
_R = _HW // _SPLIT
_N = _B * _SPLIT
_K = 12
_W = 6


def _concat_pos_kernel(x_hbm, row_ref, col_ref, o_hbm, stage, in_sems, out_sems):
    colb = jnp.broadcast_to(col_ref[...][None, :, :], (32, 32, 256)).reshape(_HW, 256)
    rowb = jnp.broadcast_to(row_ref[...][:, None, :], (32, 32, 256)).reshape(_HW, 256)
    for s in range(_K):
        lo = (s % _SPLIT) * _R
        stage[s, :, _C:_C + 256] = colb[lo:lo + _R]
        stage[s, :, _C + 256:] = rowb[lo:lo + _R]

    def in_copy(i):
        b, half = divmod(i, _SPLIT)
        return pltpu.make_async_copy(
            x_hbm.at[b, half * _R:(half + 1) * _R],
            stage.at[i % _K, :, 0:_C], in_sems.at[i % _K])

    def out_copy(i):
        b, half = divmod(i, _SPLIT)
        return pltpu.make_async_copy(
            stage.at[i % _K],
            o_hbm.at[b, half * _R:(half + 1) * _R], out_sems.at[i % _K])

    out_copies = [out_copy(i) for i in range(_N)]
    for i in range(_K):
        in_copy(i).start()
    for i in range(_N):
        in_copy(i).wait()
        out_copies[i].start()
        j = i - _W
        if j >= 0 and j + _K < _N:
            out_copies[j].wait()
            in_copy(j + _K).start()
    for i in range(_N - _K, _N):
        out_copies[i].wait()


def kernel(x, row_embed, col_embed):
    b, c, h, w = x.shape
    xt = x.transpose(0, 2, 3, 1).reshape(b, h * w, c)
    out = pl.pallas_call(
        _concat_pos_kernel,
        in_specs=[
            pl.BlockSpec(memory_space=pl.ANY),
            pl.BlockSpec(memory_space=pltpu.MemorySpace.VMEM),
            pl.BlockSpec(memory_space=pltpu.MemorySpace.VMEM),
        ],
        out_specs=pl.BlockSpec(memory_space=pl.ANY),
        out_shape=jax.ShapeDtypeStruct((b, h * w, c + _P), x.dtype),
        scratch_shapes=[
            pltpu.VMEM((_K, _R, c + _P), x.dtype),
            pltpu.SemaphoreType.DMA((_K,)),
            pltpu.SemaphoreType.DMA((_K,)),
        ],
    )(xt, row_embed, col_embed)
    return out.reshape(b, h, w, c + _P).transpose(0, 3, 1, 2)
